# Initial kernel scaffold; baseline (speedup 1.0000x reference)
import jax
import jax.numpy as jnp
from jax.experimental import pallas as pl


def kernel(colors, chars, specials, emb_colors, emb_chars, emb_specials, lin_w, lin_b):
    raise NotImplementedError("write your pallas kernel here")


# speedup vs baseline: 8.5427x; 8.5427x over previous
import functools

import jax
import jax.numpy as jnp
from jax import lax
from jax.experimental import pallas as pl
from jax.experimental.pallas import tpu as pltpu
from jax.experimental.pallas import tpu_sc as plsc

D = 64
NC = 2
NS = 16
NW = NC * NS
C = 112


def _fold_tables(emb_colors, emb_chars, emb_specials, lin_w, lin_b2d):
    def body(ec, eh, es, w, bvec, pc, ph, ps):
        pc[...] = jnp.dot(ec[...], w[0:D, :],
                          preferred_element_type=jnp.float32) + bvec[...]
        ph[...] = jnp.dot(eh[...], w[D:2 * D, :],
                          preferred_element_type=jnp.float32)
        ps[...] = jnp.dot(es[...], w[2 * D:3 * D, :],
                          preferred_element_type=jnp.float32)

    return pl.pallas_call(
        body,
        out_shape=(
            jax.ShapeDtypeStruct((16, D), jnp.float32),
            jax.ShapeDtypeStruct((256, D), jnp.float32),
            jax.ShapeDtypeStruct((256, D), jnp.float32),
        ),
    )(emb_colors, emb_chars, emb_specials, lin_w, lin_b2d)


def _sc_embed(colors, chars, specials, pc, ph, ps):
    N = colors.shape[0]
    assert N % (NW * C) == 0
    tpw = N // NW
    nch = tpw // C
    assert nch % 2 == 0

    mesh = plsc.VectorSubcoreMesh(core_axis_name="c", subcore_axis_name="s")

    @functools.partial(
        pl.kernel,
        out_type=jax.ShapeDtypeStruct((N * D,), jnp.float32),
        mesh=mesh,
        scratch_types=[
            pltpu.VMEM((16 * D,), jnp.float32),
            pltpu.VMEM((256 * D,), jnp.float32),
            pltpu.VMEM((256 * D,), jnp.float32),
            pltpu.VMEM((2, 3, C), jnp.int32),
            pltpu.SMEM((3, C), jnp.int32),
            pltpu.VMEM((2, C * D), jnp.float32),
            pltpu.SemaphoreType.DMA((2,)),
            pltpu.SemaphoreType.DMA((2,)),
        ],
    )
    def k(colors_h, chars_h, specials_h, pc_h, ph_h, ps_h, out_h,
          tabc, tabh, tabs, idxv, idx_s, outb, sem_i, sem_o):
        wid = lax.axis_index("s") * NC + lax.axis_index("c")
        base0 = wid * tpw

        pltpu.sync_copy(pc_h, tabc)
        pltpu.sync_copy(ph_h, tabh)
        pltpu.sync_copy(ps_h, tabs)

        idx_srcs = (colors_h, chars_h, specials_h)

        def start_idx(i, b):
            base = base0 + i * C
            for j, src in enumerate(idx_srcs):
                pltpu.async_copy(src.at[pl.ds(base, C)], idxv.at[b, j],
                                 sem_i.at[b])

        def wait_idx(i, b):
            base = base0 + i * C
            for j, src in enumerate(idx_srcs):
                pltpu.make_async_copy(src.at[pl.ds(base, C)], idxv.at[b, j],
                                      sem_i.at[b]).wait()

        def out_slice(i):
            return out_h.at[pl.ds((base0 + i * C) * D, C * D)]

        start_idx(0, 0)
        start_idx(1, 1)

        def outer(g, carry):
            for b in range(2):
                i = 2 * g + b
                wait_idx(i, b)

                @pl.when(i >= 2)
                def _():
                    pltpu.make_async_copy(outb.at[b], out_slice(i - 2),
                                          sem_o.at[b]).wait()

                def group_body(g2, c2):
                    t0 = g2 * 16
                    rcv = idxv[b, 0, pl.ds(t0, 16)] * D
                    rhv = idxv[b, 1, pl.ds(t0, 16)] * D
                    rsv = idxv[b, 2, pl.ds(t0, 16)] * D
                    for l in range(16):
                        rc, rh, rs = rcv[l], rhv[l], rsv[l]
                        ob = (t0 + l) * D
                        for j in range(D // 16):
                            o = 16 * j
                            v = (tabc[pl.ds(rc + o, 16)]
                                 + tabh[pl.ds(rh + o, 16)]
                                 + tabs[pl.ds(rs + o, 16)])
                            outb[b, pl.ds(ob + o, 16)] = v
                    return c2

                lax.fori_loop(0, C // 16, group_body, 0)
                pltpu.async_copy(outb.at[b], out_slice(i), sem_o.at[b])

                @pl.when(i + 2 < nch)
                def _():
                    start_idx(i + 2, b)
            return carry

        lax.fori_loop(0, nch // 2, outer, 0)
        for b in range(2):
            pltpu.make_async_copy(outb.at[b], out_slice(nch - 2 + b),
                                  sem_o.at[b]).wait()

    return k(colors, chars, specials, pc, ph, ps)


def kernel(colors, chars, specials, emb_colors, emb_chars, emb_specials,
           lin_w, lin_b):
    B, H, W = colors.shape
    N = B * H * W
    pc, ph, ps = _fold_tables(emb_colors, emb_chars, emb_specials, lin_w,
                              lin_b.reshape(1, D))
    out_flat = _sc_embed(
        colors.reshape(N), chars.reshape(N), specials.reshape(N),
        pc.reshape(16 * D), ph.reshape(256 * D), ps.reshape(256 * D))
    return out_flat.reshape(B, H, W, D)
